# baseline (speedup 1.0000x reference)
import functools

import ml_dtypes
import numpy as np
import jax
import jax.numpy as jnp
from jax import lax
from jax.experimental import pallas as pl
from jax.experimental.pallas import tpu as pltpu
from jax.experimental.pallas import tpu_sc as plsc

_B, _L, _H, _D = 2, 2048, 16, 64
_U = 40
_NH = _B * _H
_QBLK = 256
_NBLK = _L // _QBLK
_SCALE = 1.0 / np.sqrt(_D)
_NEG = np.float32(-3.0e38)

_IDX = np.asarray(
    jax.random.randint(jax.random.key(42), (_L, _U), 0, _L, dtype=jnp.int32)
)
_CNT_T = np.zeros((_L, _L), dtype=np.float32)
np.add.at(_CNT_T, (_IDX.ravel(), np.repeat(np.arange(_L), _U)), 1.0)
_BIAS_T = np.where(_CNT_T > 0.0, np.float32(0.0), _NEG)
_CNT16 = _CNT_T.T.astype(ml_dtypes.bfloat16)


def _k1_body(bias_ref, cnt16_ref, q_ref, k_ref, m_ref):
    qf = q_ref[0, 0, :, :]
    k16 = k_ref[0, 0, :, :].astype(jnp.bfloat16)
    q16 = qf.astype(jnp.bfloat16)
    ks = jax.lax.dot_general(
        cnt16_ref[:, :], k16, (((1,), (0,)), ((), ())),
        preferred_element_type=jnp.float32,
    )
    sm = jnp.sum(qf * ks, axis=1) * (1.0 / _L)
    parts = []
    for j in range(_NBLK):
        qb = q16[j * _QBLK : (j + 1) * _QBLK, :]
        st = jax.lax.dot_general(
            k16, qb, (((1,), (1,)), ((), ())),
            preferred_element_type=jnp.float32,
        )
        bb = bias_ref[:, pl.ds(j * _QBLK, _QBLK)]
        parts.append(jnp.max(st + bb, axis=0))
    m_ref[0, 0, :] = jnp.concatenate(parts, axis=0) - sm


_NC = 2
_NS = 16
_NVR = _L // 16
_UNROLL = 4
_UPAD = 48


def _sc_topk_body(m_hbm, top_hbm, mrow_v, topb_v):
    wid = lax.axis_index("s") * _NC + lax.axis_index("c")
    pltpu.sync_copy(m_hbm.at[wid], mrow_v)
    lane = lax.iota(jnp.int32, 16)
    big = jnp.int32(1 << 30)

    def smax16(vec):
        r = vec[0]
        for i in range(1, 16):
            r = jnp.maximum(r, vec[i])
        return r

    def smin16(vec):
        r = vec[0]
        for i in range(1, 16):
            r = jnp.minimum(r, vec[i])
        return r

    def lane_max(c, vbest):
        vb = vbest
        for i in range(_UNROLL):
            vb = jnp.maximum(vb, mrow_v[pl.ds((c * _UNROLL + i) * 16, 16)])
        return vb

    def extract(u, carry):
        t0, t1, t2 = carry
        vbest = lax.fori_loop(
            0, _NVR // _UNROLL, lane_max, jnp.full((16,), _NEG, jnp.float32)
        )
        gm = smax16(vbest)

        def scan(c, best):
            b = best
            for i in range(_UNROLL):
                cc = (c * _UNROLL + i) * 16 + lane
                v = mrow_v[pl.ds((c * _UNROLL + i) * 16, 16)]
                b = jnp.minimum(b, jnp.where(v == gm, cc, big))
            return b

        bvec = lax.fori_loop(0, _NVR // _UNROLL, scan,
                             jnp.full((16,), big, jnp.int32))
        gidx = smin16(bvec)
        cstar = gidx // 16
        v = mrow_v[pl.ds(cstar * 16, 16)]
        mrow_v[pl.ds(cstar * 16, 16)] = jnp.where(
            lane == gidx - cstar * 16, _NEG, v
        )
        t0 = jnp.where(lane == u, gidx, t0)
        t1 = jnp.where(lane == u - 16, gidx, t1)
        t2 = jnp.where(lane == u - 32, gidx, t2)
        return t0, t1, t2

    zeros = jnp.zeros((16,), jnp.int32)
    t0, t1, t2 = lax.fori_loop(0, _U, extract, (zeros, zeros, zeros))
    topb_v[pl.ds(0, 16)] = t0
    topb_v[pl.ds(16, 16)] = t1
    topb_v[pl.ds(32, 16)] = t2
    pltpu.sync_copy(topb_v, top_hbm.at[wid])


_sc_topk = functools.partial(
    pl.kernel,
    mesh=plsc.VectorSubcoreMesh(core_axis_name="c", subcore_axis_name="s"),
    out_type=jax.ShapeDtypeStruct((_NH, _UPAD), jnp.int32),
    scratch_types=[
        pltpu.VMEM((_L,), jnp.float32),
        pltpu.VMEM((_UPAD,), jnp.int32),
    ],
)(_sc_topk_body)


def _k3_body(top_ref, q_ref, k_ref, v_ref, o_ref):
    k = k_ref[0, 0, :, :]
    v = v_ref[0, 0, :, :]
    rows = [q_ref[0, 0, pl.ds(top_ref[0, 0, 0, u], 1), :] for u in range(_U)]
    qr = jnp.concatenate(rows, axis=0)
    s = jax.lax.dot_general(
        qr.astype(jnp.bfloat16),
        k.astype(jnp.bfloat16),
        (((1,), (1,)), ((), ())),
        preferred_element_type=jnp.float32,
    ) * _SCALE
    s = s - jnp.max(s, axis=1, keepdims=True)
    e = jnp.exp(s)
    a = e / jnp.sum(e, axis=1, keepdims=True)
    upd = jax.lax.dot_general(
        a.astype(jnp.bfloat16),
        v.astype(jnp.bfloat16),
        (((1,), (0,)), ((), ())),
        preferred_element_type=jnp.float32,
    )
    vm = jnp.mean(v, axis=0)
    o_ref[0, 0, :, :] = jnp.broadcast_to(vm[None, :], (_L, _D))
    for u in range(_U):
        o_ref[0, 0, pl.ds(top_ref[0, 0, 0, u], 1), :] = upd[u : u + 1, :]


def _qkv_spec():
    return pl.BlockSpec((1, 1, _L, _D), lambda b, h: (b, h, 0, 0))


def kernel(queries, keys, values, attn_mask):
    del attn_mask
    q = jnp.transpose(queries, (0, 2, 1, 3))
    kk = jnp.transpose(keys, (0, 2, 1, 3))
    v = jnp.transpose(values, (0, 2, 1, 3))

    m32 = pl.pallas_call(
        _k1_body,
        grid=(_B, _H),
        in_specs=[
            pl.BlockSpec((_L, _L), lambda b, h: (0, 0)),
            pl.BlockSpec((_L, _L), lambda b, h: (0, 0)),
            _qkv_spec(),
            _qkv_spec(),
        ],
        out_specs=pl.BlockSpec((1, 1, _L), lambda b, h: (b * _H + h, 0, 0)),
        out_shape=jax.ShapeDtypeStruct((_NH, 1, _L), jnp.float32),
        compiler_params=pltpu.CompilerParams(
            dimension_semantics=("arbitrary", "arbitrary"),
        ),
    )(jnp.asarray(_BIAS_T), jnp.asarray(_CNT16), q, kk)

    mtop = _sc_topk(m32.reshape(_NH, _L))[:, :_U]

    ctx = pl.pallas_call(
        _k3_body,
        grid=(_B, _H),
        in_specs=[
            pl.BlockSpec(
                (1, 1, 1, _U), lambda b, h: (b, h, 0, 0),
                memory_space=pltpu.SMEM,
            ),
            _qkv_spec(),
            _qkv_spec(),
            _qkv_spec(),
        ],
        out_specs=pl.BlockSpec((1, 1, _L, _D), lambda b, h: (b, h, 0, 0)),
        out_shape=jax.ShapeDtypeStruct((_B, _H, _L, _D), jnp.float32),
        compiler_params=pltpu.CompilerParams(
            dimension_semantics=("arbitrary", "arbitrary"),
        ),
    )(mtop.reshape(_B, _H, 1, _U), q, kk, v)
    return ctx


# speedup vs baseline: 1.5461x; 1.5461x over previous
import functools

import numpy as np
import jax
import jax.numpy as jnp
from jax import lax
from jax.experimental import pallas as pl
from jax.experimental.pallas import tpu as pltpu
from jax.experimental.pallas import tpu_sc as plsc

_B, _L, _H, _D = 2, 2048, 16, 64
_U = 40
_NH = _B * _H
_QBLK = 256
_NBLK = _L // _QBLK
_SCALE = 1.0 / np.sqrt(_D)
_NEG = np.float32(-3.0e38)

_IDX = np.asarray(
    jax.random.randint(jax.random.key(42), (_L, _U), 0, _L, dtype=jnp.int32)
)
_CNT_T = np.zeros((_L, _L), dtype=np.float32)
np.add.at(_CNT_T, (_IDX.ravel(), np.repeat(np.arange(_L), _U)), 1.0)


def _k1_body(cnt_ref, q_ref, k_ref, m_ref):
    k16 = k_ref[0, 0, :, :].astype(jnp.bfloat16)
    q16 = q_ref[0, 0, :, :].astype(jnp.bfloat16)
    for j in range(_NBLK):
        qb = q16[j * _QBLK : (j + 1) * _QBLK, :]
        st = jax.lax.dot_general(
            k16, qb, (((1,), (1,)), ((), ())),
            preferred_element_type=jnp.float32,
        )
        cb = cnt_ref[:, pl.ds(j * _QBLK, _QBLK)]
        mx = jnp.max(jnp.where(cb > 0.0, st, _NEG), axis=0)
        sm = jnp.sum(st * cb, axis=0)
        m_ref[0, 0, pl.ds(j * _QBLK, _QBLK)] = mx - sm * (1.0 / _L)


_NC = 2
_NS = 16
_NVR = _L // 16
_UNROLL = 4
_UPAD = 48


def _sc_topk_body(m_hbm, top_hbm, mrow_v, topb_v):
    wid = lax.axis_index("s") * _NC + lax.axis_index("c")
    pltpu.sync_copy(m_hbm.at[wid], mrow_v)
    lane = lax.iota(jnp.int32, 16)
    big = jnp.int32(1 << 30)

    def smax16(vec):
        r = vec[0]
        for i in range(1, 16):
            r = jnp.maximum(r, vec[i])
        return r

    def smin16(vec):
        r = vec[0]
        for i in range(1, 16):
            r = jnp.minimum(r, vec[i])
        return r

    def lane_max(c, vbest):
        vb = vbest
        for i in range(_UNROLL):
            vb = jnp.maximum(vb, mrow_v[pl.ds((c * _UNROLL + i) * 16, 16)])
        return vb

    def extract(u, carry):
        t0, t1, t2 = carry
        vbest = lax.fori_loop(
            0, _NVR // _UNROLL, lane_max, jnp.full((16,), _NEG, jnp.float32)
        )
        gm = smax16(vbest)

        def scan(c, best):
            b = best
            for i in range(_UNROLL):
                cc = (c * _UNROLL + i) * 16 + lane
                v = mrow_v[pl.ds((c * _UNROLL + i) * 16, 16)]
                b = jnp.minimum(b, jnp.where(v == gm, cc, big))
            return b

        bvec = lax.fori_loop(0, _NVR // _UNROLL, scan,
                             jnp.full((16,), big, jnp.int32))
        gidx = smin16(bvec)
        cstar = gidx // 16
        v = mrow_v[pl.ds(cstar * 16, 16)]
        mrow_v[pl.ds(cstar * 16, 16)] = jnp.where(
            lane == gidx - cstar * 16, _NEG, v
        )
        t0 = jnp.where(lane == u, gidx, t0)
        t1 = jnp.where(lane == u - 16, gidx, t1)
        t2 = jnp.where(lane == u - 32, gidx, t2)
        return t0, t1, t2

    zeros = jnp.zeros((16,), jnp.int32)
    t0, t1, t2 = lax.fori_loop(0, _U, extract, (zeros, zeros, zeros))
    topb_v[pl.ds(0, 16)] = t0
    topb_v[pl.ds(16, 16)] = t1
    topb_v[pl.ds(32, 16)] = t2
    pltpu.sync_copy(topb_v, top_hbm.at[wid])


_sc_topk = functools.partial(
    pl.kernel,
    mesh=plsc.VectorSubcoreMesh(core_axis_name="c", subcore_axis_name="s"),
    out_type=jax.ShapeDtypeStruct((_NH, _UPAD), jnp.int32),
    scratch_types=[
        pltpu.VMEM((_L,), jnp.float32),
        pltpu.VMEM((_UPAD,), jnp.int32),
    ],
)(_sc_topk_body)


def _k3_body(top_ref, q_ref, k_ref, v_ref, o_ref):
    k = k_ref[0, 0, :, :]
    v = v_ref[0, 0, :, :]
    rows = [q_ref[0, 0, pl.ds(top_ref[0, 0, 0, u], 1), :] for u in range(_U)]
    qr = jnp.concatenate(rows, axis=0)
    s = jax.lax.dot_general(
        qr.astype(jnp.bfloat16),
        k.astype(jnp.bfloat16),
        (((1,), (1,)), ((), ())),
        preferred_element_type=jnp.float32,
    ) * _SCALE
    s = s - jnp.max(s, axis=1, keepdims=True)
    e = jnp.exp(s)
    a = e / jnp.sum(e, axis=1, keepdims=True)
    upd = jax.lax.dot_general(
        a.astype(jnp.bfloat16),
        v.astype(jnp.bfloat16),
        (((1,), (0,)), ((), ())),
        preferred_element_type=jnp.float32,
    )
    vm = jnp.mean(v, axis=0)
    o_ref[0, 0, :, :] = jnp.broadcast_to(vm[None, :], (_L, _D))
    for u in range(_U):
        o_ref[0, 0, pl.ds(top_ref[0, 0, 0, u], 1), :] = upd[u : u + 1, :]


def _qkv_spec():
    return pl.BlockSpec((1, 1, _L, _D), lambda b, h: (b, h, 0, 0))


def kernel(queries, keys, values, attn_mask):
    del attn_mask
    q = jnp.transpose(queries, (0, 2, 1, 3))
    kk = jnp.transpose(keys, (0, 2, 1, 3))
    v = jnp.transpose(values, (0, 2, 1, 3))

    m32 = pl.pallas_call(
        _k1_body,
        grid=(_B, _H),
        in_specs=[
            pl.BlockSpec((_L, _L), lambda b, h: (0, 0)),
            _qkv_spec(),
            _qkv_spec(),
        ],
        out_specs=pl.BlockSpec((1, 1, _L), lambda b, h: (b * _H + h, 0, 0)),
        out_shape=jax.ShapeDtypeStruct((_NH, 1, _L), jnp.float32),
        compiler_params=pltpu.CompilerParams(
            dimension_semantics=("arbitrary", "arbitrary"),
        ),
    )(jnp.asarray(_CNT_T), q, kk)

    mtop = _sc_topk(m32.reshape(_NH, _L))[:, :_U]

    ctx = pl.pallas_call(
        _k3_body,
        grid=(_B, _H),
        in_specs=[
            pl.BlockSpec(
                (1, 1, 1, _U), lambda b, h: (b, h, 0, 0),
                memory_space=pltpu.SMEM,
            ),
            _qkv_spec(),
            _qkv_spec(),
            _qkv_spec(),
        ],
        out_specs=pl.BlockSpec((1, 1, _L, _D), lambda b, h: (b, h, 0, 0)),
        out_shape=jax.ShapeDtypeStruct((_B, _H, _L, _D), jnp.float32),
        compiler_params=pltpu.CompilerParams(
            dimension_semantics=("arbitrary", "arbitrary"),
        ),
    )(mtop.reshape(_B, _H, 1, _U), q, kk, v)
    return ctx
